# baseline (speedup 1.0000x reference)
import functools

import jax
import jax.numpy as jnp
import numpy as np
from jax.experimental import pallas as pl

_IDX = np.float32(np.float32(np.float32(85.0) / np.float32(100.0)) * np.float32(87.0))
_P = np.float32(_IDX - np.float32(73.0))
_Q = np.float32(np.float32(1.0) - _P)

_NUM_DISTINCT = 15
_D = 64


def _body(gcat_ref, z2_ref, z3_ref, w_ref, p_ref, mel_ref, out_ref):
    spec = jnp.exp(mel_ref[0])
    g = jax.lax.dot_general(
        gcat_ref[...],
        spec,
        dimension_numbers=(((1,), (0,)), ((), ())),
        precision=jax.lax.Precision.HIGHEST,
        preferred_element_type=jnp.float32,
    )
    g1 = g[:_D]
    g2 = g[_D:2 * _D]
    g3 = g[2 * _D:]
    e = (g1 * (g2 + z2_ref[...])) * (g3 + z3_ref[...])

    tcols = e.shape[1]
    neg = jnp.float32(-jnp.inf)
    wrow = w_ref[...]

    def count(mask):
        ind = jnp.where(mask, jnp.float32(1.0), jnp.float32(0.0))
        return jax.lax.dot_general(
            wrow,
            ind,
            dimension_numbers=(((1,), (0,)), ((), ())),
            preferred_element_type=jnp.float32,
        )

    nxt = jnp.max(e, axis=0, keepdims=True)
    consumed = e >= nxt
    c = count(consumed)
    s74 = jnp.where(c >= 14.0, nxt, jnp.float32(0.0))
    s73 = jnp.where(c >= 15.0, nxt, jnp.float32(0.0))
    cprev = c
    for _ in range(_NUM_DISTINCT - 1):
        masked = jnp.where(consumed, neg, e)
        nxt = jnp.max(masked, axis=0, keepdims=True)
        consumed = e >= nxt
        c = count(consumed)
        s74 = jnp.where((cprev < 14.0) & (c >= 14.0), nxt, s74)
        s73 = jnp.where((cprev < 15.0) & (c >= 15.0), nxt, s73)
        cprev = c
    thresh = (s73 * _Q) + (s74 * _P)
    cmp01 = jnp.where(e >= thresh, jnp.float32(1.0), jnp.float32(0.0)).astype(
        jnp.bfloat16
    )
    out_ref[0] = jax.lax.dot_general(
        p_ref[...],
        cmp01,
        dimension_numbers=(((1,), (0,)), ((), ())),
        preferred_element_type=jnp.float32,
    )


def kernel(mel, key_bins):
    if mel.ndim == 4:
        mel = mel[:, 0]
    b, m, t = mel.shape
    k = key_bins.shape[0]
    kb = key_bins.astype(jnp.int32)
    u = jnp.unique(kb, size=_D, fill_value=-1)
    pmat = (u[None, :] == kb[:, None]).astype(jnp.bfloat16)
    w = jnp.sum((u[None, :] == kb[:, None]).astype(jnp.float32), axis=0)[None, :]
    g1 = jax.nn.one_hot(u, m, dtype=jnp.float32)
    g2 = jax.nn.one_hot(2 * u, m, dtype=jnp.float32)
    g3 = jax.nn.one_hot(3 * u, m, dtype=jnp.float32)
    gcat = jnp.concatenate([g1, g2, g3], axis=0)
    z2 = jnp.where(2 * u < m, 0.0, 1.0).astype(jnp.float32)[:, None]
    z3 = jnp.where(3 * u < m, 0.0, 1.0).astype(jnp.float32)[:, None]
    tb = 4096
    grid = (b, t // tb)
    out = pl.pallas_call(
        _body,
        grid=grid,
        in_specs=[
            pl.BlockSpec((3 * _D, m), lambda bi, ti: (0, 0)),
            pl.BlockSpec((_D, 1), lambda bi, ti: (0, 0)),
            pl.BlockSpec((_D, 1), lambda bi, ti: (0, 0)),
            pl.BlockSpec((1, _D), lambda bi, ti: (0, 0)),
            pl.BlockSpec((k, _D), lambda bi, ti: (0, 0)),
            pl.BlockSpec((1, m, tb), lambda bi, ti: (bi, 0, ti)),
        ],
        out_specs=pl.BlockSpec((1, k, tb), lambda bi, ti: (bi, 0, ti)),
        out_shape=jax.ShapeDtypeStruct((b, k, t), jnp.float32),
    )(gcat, z2, z3, w, pmat, mel)
    return (out, out)


# speedup vs baseline: 1.2127x; 1.2127x over previous
import functools

import jax
import jax.numpy as jnp
import numpy as np
from jax.experimental import pallas as pl

_IDX = np.float32(np.float32(np.float32(85.0) / np.float32(100.0)) * np.float32(87.0))
_P = np.float32(_IDX - np.float32(73.0))
_Q = np.float32(np.float32(1.0) - _P)

_NUM_DISTINCT = 15
_D = 64


def _body(gcat_ref, z2_ref, z3_ref, w_ref, p_ref, mel_ref, out_ref, out2_ref):
    spec = jnp.exp(mel_ref[0])
    g = jax.lax.dot_general(
        gcat_ref[...],
        spec,
        dimension_numbers=(((1,), (0,)), ((), ())),
        precision=jax.lax.Precision.HIGHEST,
        preferred_element_type=jnp.float32,
    )
    g1 = g[:_D]
    g2 = g[_D:2 * _D]
    g3 = g[2 * _D:]
    e = (g1 * (g2 + z2_ref[...])) * (g3 + z3_ref[...])

    tcols = e.shape[1]
    neg = jnp.float32(-jnp.inf)
    wrow = w_ref[...]

    def count(mask):
        ind = jnp.where(mask, jnp.float32(1.0), jnp.float32(0.0))
        return jax.lax.dot_general(
            wrow,
            ind,
            dimension_numbers=(((1,), (0,)), ((), ())),
            preferred_element_type=jnp.float32,
        )

    nxt = jnp.max(e, axis=0, keepdims=True)
    consumed = e >= nxt
    c = count(consumed)
    s74 = jnp.where(c >= 14.0, nxt, jnp.float32(0.0))
    s73 = jnp.where(c >= 15.0, nxt, jnp.float32(0.0))
    cprev = c
    for _ in range(_NUM_DISTINCT - 1):
        masked = jnp.where(consumed, neg, e)
        nxt = jnp.max(masked, axis=0, keepdims=True)
        consumed = e >= nxt
        c = count(consumed)
        s74 = jnp.where((cprev < 14.0) & (c >= 14.0), nxt, s74)
        s73 = jnp.where((cprev < 15.0) & (c >= 15.0), nxt, s73)
        cprev = c
    thresh = (s73 * _Q) + (s74 * _P)
    cmp01 = jnp.where(e >= thresh, jnp.float32(1.0), jnp.float32(0.0)).astype(
        jnp.bfloat16
    )
    probs = jax.lax.dot_general(
        p_ref[...],
        cmp01,
        dimension_numbers=(((1,), (0,)), ((), ())),
        preferred_element_type=jnp.float32,
    )
    out_ref[0] = probs
    out2_ref[0] = probs


def kernel(mel, key_bins):
    if mel.ndim == 4:
        mel = mel[:, 0]
    b, m, t = mel.shape
    k = key_bins.shape[0]
    kb = key_bins.astype(jnp.int32)
    first = jnp.concatenate([jnp.ones((1,), jnp.int32), (kb[1:] != kb[:-1]).astype(jnp.int32)])
    seg = jnp.cumsum(first) - 1
    pmat_f = (seg[:, None] == jnp.arange(_D)[None, :]).astype(jnp.float32)
    pmat = pmat_f.astype(jnp.bfloat16)
    w = jnp.sum(pmat_f, axis=0)[None, :]
    u = jnp.max(pmat_f * (kb[:, None] + 1).astype(jnp.float32), axis=0).astype(jnp.int32) - 1
    g1 = jax.nn.one_hot(u, m, dtype=jnp.float32)
    g2 = jax.nn.one_hot(2 * u, m, dtype=jnp.float32)
    g3 = jax.nn.one_hot(3 * u, m, dtype=jnp.float32)
    gcat = jnp.concatenate([g1, g2, g3], axis=0)
    z2 = jnp.where((2 * u < m) & (u >= 0), 0.0, 1.0).astype(jnp.float32)[:, None]
    z3 = jnp.where((3 * u < m) & (u >= 0), 0.0, 1.0).astype(jnp.float32)[:, None]
    tb = 4096
    grid = (b, t // tb)
    out = pl.pallas_call(
        _body,
        grid=grid,
        in_specs=[
            pl.BlockSpec((3 * _D, m), lambda bi, ti: (0, 0)),
            pl.BlockSpec((_D, 1), lambda bi, ti: (0, 0)),
            pl.BlockSpec((_D, 1), lambda bi, ti: (0, 0)),
            pl.BlockSpec((1, _D), lambda bi, ti: (0, 0)),
            pl.BlockSpec((k, _D), lambda bi, ti: (0, 0)),
            pl.BlockSpec((1, m, tb), lambda bi, ti: (bi, 0, ti)),
        ],
        out_specs=[
            pl.BlockSpec((1, k, tb), lambda bi, ti: (bi, 0, ti)),
            pl.BlockSpec((1, k, tb), lambda bi, ti: (bi, 0, ti)),
        ],
        out_shape=[
            jax.ShapeDtypeStruct((b, k, t), jnp.float32),
            jax.ShapeDtypeStruct((b, k, t), jnp.float32),
        ],
    )(gcat, z2, z3, w, pmat, mel)
    return (out[0], out[1])


# speedup vs baseline: 1.2140x; 1.0011x over previous
import jax
import jax.numpy as jnp
import numpy as np
from jax.experimental import pallas as pl

_IDX = np.float32(np.float32(np.float32(85.0) / np.float32(100.0)) * np.float32(87.0))
_P = np.float32(_IDX - np.float32(73.0))
_Q = np.float32(np.float32(1.0) - _P)

_NUM_DISTINCT = 15
_D = 64


def _body(gcat_ref, z2_ref, z3_ref, w_ref, p_ref, mel_ref, out_ref, out2_ref):
    spec = jnp.exp(mel_ref[0])
    g = jax.lax.dot_general(
        gcat_ref[...],
        spec,
        dimension_numbers=(((1,), (0,)), ((), ())),
        precision=jax.lax.Precision.HIGHEST,
        preferred_element_type=jnp.float32,
    )
    g1 = g[:_D]
    g2 = g[_D:2 * _D]
    g3 = g[2 * _D:]
    e = (g1 * (g2 + z2_ref[...])) * (g3 + z3_ref[...])

    tcols = e.shape[1]
    neg = jnp.float32(-jnp.inf)
    wrow = w_ref[...]

    def count(mask):
        ind = jnp.where(mask, jnp.float32(1.0), jnp.float32(0.0))
        return jax.lax.dot_general(
            wrow,
            ind,
            dimension_numbers=(((1,), (0,)), ((), ())),
            preferred_element_type=jnp.float32,
        )

    nxt = jnp.max(e, axis=0, keepdims=True)
    consumed = e >= nxt
    c = count(consumed)
    s74 = jnp.where(c >= 14.0, nxt, jnp.float32(0.0))
    s73 = jnp.where(c >= 15.0, nxt, jnp.float32(0.0))
    cprev = c
    for _ in range(_NUM_DISTINCT - 1):
        masked = jnp.where(consumed, neg, e)
        nxt = jnp.max(masked, axis=0, keepdims=True)
        consumed = e >= nxt
        c = count(consumed)
        s74 = jnp.where((cprev < 14.0) & (c >= 14.0), nxt, s74)
        s73 = jnp.where((cprev < 15.0) & (c >= 15.0), nxt, s73)
        cprev = c
    thresh = (s73 * _Q) + (s74 * _P)
    cmp01 = jnp.where(e >= thresh, jnp.float32(1.0), jnp.float32(0.0)).astype(
        jnp.bfloat16
    )
    probs = jax.lax.dot_general(
        p_ref[...],
        cmp01,
        dimension_numbers=(((1,), (0,)), ((), ())),
        preferred_element_type=jnp.float32,
    )
    out_ref[0] = probs
    out2_ref[0] = probs


def kernel(mel, key_bins):
    if mel.ndim == 4:
        mel = mel[:, 0]
    b, m, t = mel.shape
    k = key_bins.shape[0]
    kb = key_bins.astype(jnp.int32)
    first = jnp.concatenate([jnp.ones((1,), jnp.int32), (kb[1:] != kb[:-1]).astype(jnp.int32)])
    seg = jnp.cumsum(first) - 1
    pmat_f = (seg[:, None] == jnp.arange(_D)[None, :]).astype(jnp.float32)
    pmat = pmat_f.astype(jnp.bfloat16)
    w = jnp.sum(pmat_f, axis=0)[None, :]
    u = jnp.max(pmat_f * (kb[:, None] + 1).astype(jnp.float32), axis=0).astype(jnp.int32) - 1
    g1 = jax.nn.one_hot(u, m, dtype=jnp.float32)
    g2 = jax.nn.one_hot(2 * u, m, dtype=jnp.float32)
    g3 = jax.nn.one_hot(3 * u, m, dtype=jnp.float32)
    gcat = jnp.concatenate([g1, g2, g3], axis=0)
    z2 = jnp.where((2 * u < m) & (u >= 0), 0.0, 1.0).astype(jnp.float32)[:, None]
    z3 = jnp.where((3 * u < m) & (u >= 0), 0.0, 1.0).astype(jnp.float32)[:, None]
    tb = 4096
    grid = (b, t // tb)
    out = pl.pallas_call(
        _body,
        grid=grid,
        in_specs=[
            pl.BlockSpec((3 * _D, m), lambda bi, ti: (0, 0)),
            pl.BlockSpec((_D, 1), lambda bi, ti: (0, 0)),
            pl.BlockSpec((_D, 1), lambda bi, ti: (0, 0)),
            pl.BlockSpec((1, _D), lambda bi, ti: (0, 0)),
            pl.BlockSpec((k, _D), lambda bi, ti: (0, 0)),
            pl.BlockSpec((1, m, tb), lambda bi, ti: (bi, 0, ti)),
        ],
        out_specs=[
            pl.BlockSpec((1, k, tb), lambda bi, ti: (bi, 0, ti)),
            pl.BlockSpec((1, k, tb), lambda bi, ti: (bi, 0, ti)),
        ],
        out_shape=[
            jax.ShapeDtypeStruct((b, k, t), jnp.float32),
            jax.ShapeDtypeStruct((b, k, t), jnp.float32),
        ],
    )(gcat, z2, z3, w, pmat, mel)
    return (out[0], out[1])
